# baseline (speedup 1.0000x reference)
import functools

import jax
import jax.numpy as jnp
from jax import lax
from jax.experimental import pallas as pl
from jax.experimental.pallas import tpu as pltpu
from jax.experimental.pallas import tpu_sc as plsc

_VOCAB = 1000000
_D = 32
_B = 16384
_W = 512
_LASTFULL = 1952
_TAILW = 1953
_TAIL_BASE = _TAILW * _W
_NSLOT = 62
_PAD_ROW = _B


def _make_rc():
    info = plsc.get_sparse_core_info()
    nc = info.num_cores
    nw = nc * info.num_subcores
    nv = _B // 16

    mesh = plsc.VectorSubcoreMesh(core_axis_name="c", subcore_axis_name="s")

    @functools.partial(
        pl.kernel,
        mesh=mesh,
        out_type=jax.ShapeDtypeStruct((_B * _D + 16 * _D,), jnp.float32),
        scratch_types=[
            pltpu.VMEM((_B,), jnp.int32),
            pltpu.VMEM((_B + 16,), jnp.int32),
            pltpu.VMEM((_B + 16,), jnp.int32),
            pltpu.VMEM((_D, _W), jnp.float32),
            pltpu.VMEM((_D, _W), jnp.float32),
            pltpu.VMEM((_B + 16,), jnp.int32),
            pltpu.VMEM((_B + 16,), jnp.int32),
            pltpu.VMEM((16, _D), jnp.float32),
            pltpu.SemaphoreType.DMA,
            pltpu.SemaphoreType.DMA,
            pltpu.SemaphoreType.DMA,
        ],
        compiler_params=pltpu.CompilerParams(needs_layout_passes=False),
    )
    def rc_kernel(x_hbm, tt_hbm, tail_hbm, out_hbm,
                  xbuf, hx, hp, win0, win1, wl, wp, stage,
                  ssem0, ssem1, dsem):
        wid = lax.axis_index("s") * nc + lax.axis_index("c")
        iota16 = lax.iota(jnp.int32, 16)

        pltpu.sync_copy(x_hbm, xbuf)

        def append(dst_a, dst_b, off, va, vb, sel):
            m = plsc.all_reduce_population_count(sel)[0]

            @pl.when(m > 0)
            def _():
                keys = jnp.where(sel, iota16, iota16 + 16)
                _, sa = plsc.sort_key_val(keys, va)
                _, sb = plsc.sort_key_val(keys, vb)
                dst_a[pl.ds(off, 16)] = sa
                dst_b[pl.ds(off, 16)] = sb

            return m

        def scan_body(v, cnt):
            xv = xbuf[pl.ds(v * 16, 16)]
            sel = ((xv >> 9) & (nw - 1)) == wid
            m = append(hx, hp, cnt, xv, iota16 + v * 16, sel)
            return cnt + m

        cnt = lax.fori_loop(0, nv, scan_body, jnp.int32(0))
        nhv = (cnt + 15) >> 4

        def win_slice(slot):
            w_id = jnp.minimum(slot * nw + wid, _LASTFULL)
            return tt_hbm.at[:, pl.ds(w_id * _W, _W)]

        def fire(slot, buf, sem):
            @pl.when(slot < _NSLOT)
            def _():
                pltpu.async_copy(win_slice(slot), buf, sem)

        def process(slot, buf, sem):
            w_id = slot * nw + wid
            pltpu.make_async_copy(win_slice(slot), buf, sem).wait()
            lo = w_id * _W

            def rescan(h, mcnt):
                xv = hx[pl.ds(h * 16, 16)]
                pv = hp[pl.ds(h * 16, 16)]
                sel = (xv >> 9) == w_id
                m = append(wl, wp, mcnt, xv - lo, pv, sel)
                return mcnt + m

            mcnt = lax.fori_loop(0, nhv, rescan, jnp.int32(0))

            @pl.when(w_id <= _LASTFULL)
            def _full():
                def chunk(q, _):
                    rem = mcnt - q * 16
                    keep = iota16 < rem
                    lv = jnp.where(keep, wl[pl.ds(q * 16, 16)], 0)
                    pv = jnp.where(keep, wp[pl.ds(q * 16, 16)], _PAD_ROW)
                    for lane in range(16):
                        l_s = lv[lane]
                        lsp = jnp.full((16,), l_s, jnp.int32)
                        stage[lane, pl.ds(0, 16)] = plsc.load_gather(
                            buf, [iota16, lsp])
                        stage[lane, pl.ds(16, 16)] = plsc.load_gather(
                            buf, [iota16 + 16, lsp])
                    copies = [
                        pltpu.async_copy(
                            stage.at[lane],
                            out_hbm.at[pl.ds(pv[lane] * _D, _D)], dsem)
                        for lane in range(16)
                    ]
                    for c in copies:
                        c.wait()
                    return ()

                lax.fori_loop(0, (mcnt + 15) >> 4, chunk, ())

            @pl.when(w_id == _TAILW)
            def _tail():
                def chunk(q, _):
                    rem = mcnt - q * 16
                    lv = wl[pl.ds(q * 16, 16)]
                    pv = wp[pl.ds(q * 16, 16)]
                    for lane in range(16):
                        @pl.when(lane < rem)
                        def _one():
                            pltpu.sync_copy(
                                tail_hbm.at[pl.ds(lv[lane] * _D, _D)],
                                stage.at[lane])
                            pltpu.sync_copy(
                                stage.at[lane],
                                out_hbm.at[pl.ds(pv[lane] * _D, _D)])
                    return ()

                lax.fori_loop(0, (mcnt + 15) >> 4, chunk, ())

        fire(jnp.int32(0), win0, ssem0)
        fire(jnp.int32(1), win1, ssem1)

        def wloop(k, _):
            s0 = k * 2
            process(s0, win0, ssem0)
            fire(s0 + 2, win0, ssem0)
            process(s0 + 1, win1, ssem1)
            fire(s0 + 3, win1, ssem1)
            return ()

        lax.fori_loop(0, _NSLOT // 2, wloop, ())

    return rc_kernel


def kernel(x, eps_table):
    tt = eps_table.T
    tail = eps_table[_TAIL_BASE:, :].reshape(-1)
    out1d = _make_rc()(x, tt, tail)
    return out1d[: _B * _D].reshape(_B, _D)


# speedup vs baseline: 2.1202x; 2.1202x over previous
import functools

import jax
import jax.numpy as jnp
from jax import lax
from jax.experimental import pallas as pl
from jax.experimental.pallas import tpu as pltpu
from jax.experimental.pallas import tpu_sc as plsc

_VOCAB = 1000000
_D = 32
_B = 16384
_W = 512
_LASTFULL = 1952
_TAILW = 1953
_TAIL_BASE = _TAILW * _W
_NSLOT = 62
_PAD_ROW = _B
_FCAP = 256


def _make_rc():
    info = plsc.get_sparse_core_info()
    nc = info.num_cores
    nw = nc * info.num_subcores
    nv = _B // 16

    mesh = plsc.VectorSubcoreMesh(core_axis_name="c", subcore_axis_name="s")

    @functools.partial(
        pl.kernel,
        mesh=mesh,
        out_type=jax.ShapeDtypeStruct((_B * _D + 16 * _D,), jnp.float32),
        scratch_types=[
            pltpu.VMEM((_B,), jnp.int32),
            pltpu.VMEM((_B + 16,), jnp.int32),
            pltpu.VMEM((_B + 16,), jnp.int32),
            pltpu.VMEM((_D, _W), jnp.float32),
            pltpu.VMEM((_D, _W), jnp.float32),
            pltpu.VMEM((_B + 16,), jnp.int32),
            pltpu.VMEM((_B + 16,), jnp.int32),
            pltpu.VMEM(((_FCAP + 16) * _D,), jnp.float32),
            pltpu.VMEM((_FCAP + 16,), jnp.int32),
            pltpu.SemaphoreType.DMA,
            pltpu.SemaphoreType.DMA,
            pltpu.SemaphoreType.DMA,
        ],
        compiler_params=pltpu.CompilerParams(needs_layout_passes=False),
    )
    def rc_kernel(x_hbm, tt_hbm, tail_hbm, out_hbm,
                  xbuf, hx, hp, win0, win1, wl, wp, outf, opl,
                  ssem0, ssem1, dsem):
        wid = lax.axis_index("s") * nc + lax.axis_index("c")
        iota16 = lax.iota(jnp.int32, 16)

        pltpu.sync_copy(x_hbm, xbuf)

        def append(dst_a, dst_b, off, va, vb, sel):
            m = plsc.all_reduce_population_count(sel)[0]

            @pl.when(m > 0)
            def _():
                keys = jnp.where(sel, iota16, iota16 + 16)
                _, sa = plsc.sort_key_val(keys, va)
                _, sb = plsc.sort_key_val(keys, vb)
                dst_a[pl.ds(off, 16)] = sa
                dst_b[pl.ds(off, 16)] = sb

            return m

        def scan_body(v, cnt):
            xv = xbuf[pl.ds(v * 16, 16)]
            sel = ((xv >> 9) & (nw - 1)) == wid
            m = append(hx, hp, cnt, xv, iota16 + v * 16, sel)
            return cnt + m

        cnt = lax.fori_loop(0, nv, scan_body, jnp.int32(0))
        nhv = (cnt + 15) >> 4

        def win_slice(slot):
            w_id = jnp.minimum(slot * nw + wid, _LASTFULL)
            return tt_hbm.at[:, pl.ds(w_id * _W, _W)]

        def fire(slot, buf, sem):
            @pl.when(slot < _NSLOT)
            def _():
                pltpu.async_copy(win_slice(slot), buf, sem)

        def flush(n):
            def fbody(r, _):
                opv = opl[pl.ds(r * 16, 16)]
                for lane in range(16):
                    @pl.when(r * 16 + lane < n)
                    def _():
                        pltpu.async_copy(
                            outf.at[pl.ds((r * 16 + lane) * _D, _D)],
                            out_hbm.at[pl.ds(opv[lane] * _D, _D)], dsem)
                return ()

            lax.fori_loop(0, (n + 15) >> 4, fbody, ())

            def dbody(r, _):
                pltpu.make_async_copy(
                    outf.at[pl.ds(0, _D)],
                    out_hbm.at[pl.ds(_PAD_ROW * _D, _D)], dsem).wait()
                return ()

            lax.fori_loop(0, n, dbody, ())

        def process(slot, buf, sem, rowcnt):
            w_id = slot * nw + wid
            pltpu.make_async_copy(win_slice(slot), buf, sem).wait()
            lo = w_id * _W

            def rescan(h, mcnt):
                xv = hx[pl.ds(h * 16, 16)]
                pv = hp[pl.ds(h * 16, 16)]
                sel = (xv >> 9) == w_id
                m = append(wl, wp, mcnt, xv - lo, pv, sel)
                return mcnt + m

            mcnt = lax.fori_loop(0, nhv, rescan, jnp.int32(0))

            def chunk(q, rc):
                rem = mcnt - q * 16
                keep = iota16 < rem
                lv = jnp.where(keep, wl[pl.ds(q * 16, 16)], 0)
                pv = jnp.where(keep, wp[pl.ds(q * 16, 16)], _PAD_ROW)
                opl[pl.ds(rc, 16)] = pv

                @pl.when(w_id != _TAILW)
                def _full():
                    for lane in range(16):
                        lsp = jnp.full((16,), lv[lane], jnp.int32)
                        outf[pl.ds((rc + lane) * _D, 16)] = plsc.load_gather(
                            buf, [iota16, lsp])
                        outf[pl.ds((rc + lane) * _D + 16, 16)] = (
                            plsc.load_gather(buf, [iota16 + 16, lsp]))

                @pl.when(w_id == _TAILW)
                def _tail():
                    for lane in range(16):
                        @pl.when(lane < rem)
                        def _one():
                            pltpu.sync_copy(
                                tail_hbm.at[pl.ds(lv[lane] * _D, _D)],
                                outf.at[pl.ds((rc + lane) * _D, _D)])

                rc = rc + jnp.minimum(rem, 16)
                do_flush = rc >= _FCAP

                @pl.when(do_flush)
                def _():
                    flush(rc)

                return jnp.where(do_flush, 0, rc)

            return lax.fori_loop(0, (mcnt + 15) >> 4, chunk, rowcnt)

        fire(jnp.int32(0), win0, ssem0)
        fire(jnp.int32(1), win1, ssem1)

        def wloop(k, rowcnt):
            s0 = k * 2
            rowcnt = process(s0, win0, ssem0, rowcnt)
            fire(s0 + 2, win0, ssem0)
            rowcnt = process(s0 + 1, win1, ssem1, rowcnt)
            fire(s0 + 3, win1, ssem1)
            return rowcnt

        rowcnt = lax.fori_loop(0, _NSLOT // 2, wloop, jnp.int32(0))

        @pl.when(rowcnt > 0)
        def _():
            flush(rowcnt)

    return rc_kernel


def kernel(x, eps_table):
    tt = eps_table.T
    tail = eps_table[_TAIL_BASE:, :].reshape(-1)
    out1d = _make_rc()(x, tt, tail)
    return out1d[: _B * _D].reshape(_B, _D)
